# Initial kernel scaffold; baseline (speedup 1.0000x reference)
import jax
import jax.numpy as jnp
from jax.experimental import pallas as pl


def kernel(h3_5, h3_6, h3_7, h3_8, s2_11, s2_13, s2_14, s2_15, tab0, tab1, tab2, tab3, tab4, tab5, tab6, tab7, W, b, gamma, beta):
    raise NotImplementedError("write your pallas kernel here")


# speedup vs baseline: 24.7077x; 24.7077x over previous
import functools

import jax
import jax.numpy as jnp
from jax import lax
from jax.experimental import pallas as pl
from jax.experimental.pallas import tpu as pltpu
from jax.experimental.pallas import tpu_sc as plsc

_B, _L = 16384, 50
_N = _B * _L
_D = 128
_DL = 16
_NT = 8

_NC, _NS = 2, 16
_NW = _NC * _NS
_TPW = _N // _NW
_C = 128
_G = _TPW // _C


def _sc_gather(idx_all, tabs):
    mesh = plsc.VectorSubcoreMesh(core_axis_name="c", subcore_axis_name="s")

    @functools.partial(
        pl.kernel,
        out_type=jax.ShapeDtypeStruct((_N, _D), jnp.float32),
        mesh=mesh,
        scratch_types=[
            pltpu.VMEM((_NT, _C), jnp.int32),
            pltpu.VMEM((_NT, _C, _DL), jnp.float32),
            pltpu.SemaphoreType.DMA((_NT,)),
        ],
        compiler_params=pltpu.CompilerParams(use_tc_tiling_on_sc=False),
    )
    def k(idx_hbm, t0, t1, t2, t3, t4, t5, t6, t7, out_hbm, idx_v, rows_v,
          sems):
        tab_refs = (t0, t1, t2, t3, t4, t5, t6, t7)
        wid = lax.axis_index("s") * _NC + lax.axis_index("c")
        wbase = wid * _TPW

        def body(g, carry):
            base = wbase + g * _C
            pltpu.sync_copy(idx_hbm.at[:, pl.ds(base, _C)], idx_v)
            for t in range(_NT):
                pltpu.async_copy(tab_refs[t].at[idx_v.at[t]], rows_v.at[t],
                                 sems.at[t])
            for t in range(_NT):
                pltpu.make_async_copy(tab_refs[t].at[idx_v.at[t]],
                                      rows_v.at[t], sems.at[t]).wait()
            for t in range(_NT):
                pltpu.sync_copy(
                    rows_v.at[t],
                    out_hbm.at[pl.ds(base, _C), pl.ds(t * _DL, _DL)])
            return carry

        lax.fori_loop(0, _G, body, 0)

    return k(idx_all, *tabs)


def _tc_fuse(combined, w, b2, g2, be2):
    nb = 2048

    def body(c_ref, w_ref, b_ref, g_ref, be_ref, o_ref):
        x = lax.dot_general(c_ref[...], w_ref[...], (((1,), (1,)), ((), ())),
                            preferred_element_type=jnp.float32)
        x = x + b_ref[...]
        mean = jnp.mean(x, axis=-1, keepdims=True)
        xc = x - mean
        var = jnp.mean(xc * xc, axis=-1, keepdims=True)
        xn = xc * lax.rsqrt(var + 1e-5)
        o_ref[...] = xn * g_ref[...] + be_ref[...]

    return pl.pallas_call(
        body,
        grid=(_N // nb,),
        in_specs=[
            pl.BlockSpec((nb, _D), lambda i: (i, 0)),
            pl.BlockSpec((_D, _D), lambda i: (0, 0)),
            pl.BlockSpec((1, _D), lambda i: (0, 0)),
            pl.BlockSpec((1, _D), lambda i: (0, 0)),
            pl.BlockSpec((1, _D), lambda i: (0, 0)),
        ],
        out_specs=pl.BlockSpec((nb, _D), lambda i: (i, 0)),
        out_shape=jax.ShapeDtypeStruct((_N, _D), jnp.float32),
    )(combined, w, b2, g2, be2)


def kernel(h3_5, h3_6, h3_7, h3_8, s2_11, s2_13, s2_14, s2_15,
           tab0, tab1, tab2, tab3, tab4, tab5, tab6, tab7,
           W, b, gamma, beta):
    idx_all = jnp.stack([a.reshape(_N) for a in
                         (h3_5, h3_6, h3_7, h3_8, s2_11, s2_13, s2_14,
                          s2_15)])
    combined = _sc_gather(idx_all, (tab0, tab1, tab2, tab3, tab4, tab5,
                                    tab6, tab7))
    out = _tc_fuse(combined, W, b.reshape(1, _D), gamma.reshape(1, _D),
                   beta.reshape(1, _D))
    return out.reshape(_B, _L, _D)


# speedup vs baseline: 25.9233x; 1.0492x over previous
import functools

import jax
import jax.numpy as jnp
from jax import lax
from jax.experimental import pallas as pl
from jax.experimental.pallas import tpu as pltpu
from jax.experimental.pallas import tpu_sc as plsc

_B, _L = 16384, 50
_N = _B * _L
_D = 128
_DL = 16
_NT = 8

_NC, _NS = 2, 16
_NW = _NC * _NS
_TPW = _N // _NW
_C = 128
_G = _TPW // _C
_GC = 8


def _sc_gather(idx_list, tabs):
    mesh = plsc.VectorSubcoreMesh(core_axis_name="c", subcore_axis_name="s")

    @functools.partial(
        pl.kernel,
        out_type=jax.ShapeDtypeStruct((_N, _D), jnp.float32),
        mesh=mesh,
        scratch_types=[
            pltpu.VMEM((_NT, _GC, _C), jnp.int32),
            pltpu.VMEM((_NT, _C, _DL), jnp.float32),
            pltpu.SemaphoreType.DMA((_NT,)),
        ],
        compiler_params=pltpu.CompilerParams(use_tc_tiling_on_sc=False),
    )
    def k(i0, i1, i2, i3, i4, i5, i6, i7, t0, t1, t2, t3, t4, t5, t6, t7,
          out_hbm, idx_v, rows_v, gsem):
        idx_refs = (i0, i1, i2, i3, i4, i5, i6, i7)
        tab_refs = (t0, t1, t2, t3, t4, t5, t6, t7)
        wid = lax.axis_index("s") * _NC + lax.axis_index("c")
        wrow = wid * _G

        def body(j, carry):
            g = j // _GC
            jc = j % _GC

            @pl.when(jc == 0)
            def _idx():
                for t in range(_NT):
                    pltpu.sync_copy(
                        idx_refs[t].at[pl.ds(wrow + g * _GC, _GC)],
                        idx_v.at[t])

            for t in range(_NT):
                pltpu.async_copy(tab_refs[t].at[idx_v.at[t, jc]],
                                 rows_v.at[t], gsem.at[t])
            for t in range(_NT):
                pltpu.make_async_copy(tab_refs[t].at[idx_v.at[t, jc]],
                                      rows_v.at[t], gsem.at[t]).wait()
            base = (wid * _TPW) + j * _C
            for t in range(_NT):
                pltpu.sync_copy(
                    rows_v.at[t],
                    out_hbm.at[pl.ds(base, _C), pl.ds(t * _DL, _DL)])
            return carry

        lax.fori_loop(0, _G, body, 0)

    return k(*idx_list, *tabs)


def _tc_fuse(combined, w, b2, g2, be2):
    nb = 2048

    def body(c_ref, w_ref, b_ref, g_ref, be_ref, o_ref):
        x = lax.dot_general(c_ref[...], w_ref[...], (((1,), (1,)), ((), ())),
                            preferred_element_type=jnp.float32)
        x = x + b_ref[...]
        mean = jnp.mean(x, axis=-1, keepdims=True)
        xc = x - mean
        var = jnp.mean(xc * xc, axis=-1, keepdims=True)
        xn = xc * lax.rsqrt(var + 1e-5)
        o_ref[...] = xn * g_ref[...] + be_ref[...]

    return pl.pallas_call(
        body,
        grid=(_N // nb,),
        in_specs=[
            pl.BlockSpec((nb, _D), lambda i: (i, 0)),
            pl.BlockSpec((_D, _D), lambda i: (0, 0)),
            pl.BlockSpec((1, _D), lambda i: (0, 0)),
            pl.BlockSpec((1, _D), lambda i: (0, 0)),
            pl.BlockSpec((1, _D), lambda i: (0, 0)),
        ],
        out_specs=pl.BlockSpec((nb, _D), lambda i: (i, 0)),
        out_shape=jax.ShapeDtypeStruct((_N, _D), jnp.float32),
    )(combined, w, b2, g2, be2)


def kernel(h3_5, h3_6, h3_7, h3_8, s2_11, s2_13, s2_14, s2_15,
           tab0, tab1, tab2, tab3, tab4, tab5, tab6, tab7,
           W, b, gamma, beta):
    idx_list = [a.reshape(_N // _C, _C) for a in
                (h3_5, h3_6, h3_7, h3_8, s2_11, s2_13, s2_14, s2_15)]
    combined = _sc_gather(idx_list, (tab0, tab1, tab2, tab3, tab4, tab5,
                                     tab6, tab7))
    out = _tc_fuse(combined, W, b.reshape(1, _D), gamma.reshape(1, _D),
                   beta.reshape(1, _D))
    return out.reshape(_B, _L, _D)


# speedup vs baseline: 26.5238x; 1.0232x over previous
import functools

import jax
import jax.numpy as jnp
from jax import lax
from jax.experimental import pallas as pl
from jax.experimental.pallas import tpu as pltpu
from jax.experimental.pallas import tpu_sc as plsc

_B, _L = 16384, 50
_N = _B * _L
_D = 128
_DL = 16
_NT = 8

_NC, _NS = 2, 16
_NW = _NC * _NS
_TPW = _N // _NW
_C = 128
_G = _TPW // _C
_GC = 25


def _tc_stage_idx(idx_arrs):
    nr = 800

    def body(*refs):
        for t in range(_NT):
            refs[_NT + t][...] = refs[t][...]

    return pl.pallas_call(
        body,
        grid=(_N // _C // nr,),
        in_specs=[pl.BlockSpec((nr, _C), lambda i: (i, 0))] * _NT,
        out_specs=[pl.BlockSpec((nr, _C), lambda i: (i, 0))] * _NT,
        out_shape=[jax.ShapeDtypeStruct((_N // _C, _C), jnp.int32)] * _NT,
    )(*[a.reshape(_N // _C, _C) for a in idx_arrs])


def _sc_gather(idx_list, tabs):
    mesh = plsc.VectorSubcoreMesh(core_axis_name="c", subcore_axis_name="s")

    @functools.partial(
        pl.kernel,
        out_type=jax.ShapeDtypeStruct((_N, _D), jnp.float32),
        mesh=mesh,
        scratch_types=[
            pltpu.VMEM((_NT, _GC, _C), jnp.int32),
            pltpu.VMEM((_NT, _C, _DL), jnp.float32),
            pltpu.SemaphoreType.DMA((_NT,)),
        ],
        compiler_params=pltpu.CompilerParams(use_tc_tiling_on_sc=False),
    )
    def k(i0, i1, i2, i3, i4, i5, i6, i7, t0, t1, t2, t3, t4, t5, t6, t7,
          out_hbm, idx_v, rows_v, gsem):
        idx_refs = (i0, i1, i2, i3, i4, i5, i6, i7)
        tab_refs = (t0, t1, t2, t3, t4, t5, t6, t7)
        wid = lax.axis_index("s") * _NC + lax.axis_index("c")
        wrow = wid * _G

        def body(j, carry):
            g = j // _GC
            jc = j % _GC

            @pl.when(jc == 0)
            def _idx():
                for t in range(_NT):
                    pltpu.sync_copy(
                        idx_refs[t].at[pl.ds(wrow + g * _GC, _GC)],
                        idx_v.at[t])

            for t in range(_NT):
                pltpu.async_copy(tab_refs[t].at[idx_v.at[t, jc]],
                                 rows_v.at[t], gsem.at[t])
            for t in range(_NT):
                pltpu.make_async_copy(tab_refs[t].at[idx_v.at[t, jc]],
                                      rows_v.at[t], gsem.at[t]).wait()
            base = (wid * _TPW) + j * _C
            for t in range(_NT):
                pltpu.sync_copy(
                    rows_v.at[t],
                    out_hbm.at[pl.ds(base, _C), pl.ds(t * _DL, _DL)])
            return carry

        lax.fori_loop(0, _G, body, 0)

    return k(*idx_list, *tabs)


def _tc_fuse(combined, w, b2, g2, be2):
    nb = 2048

    def body(c_ref, w_ref, b_ref, g_ref, be_ref, o_ref):
        x = lax.dot_general(c_ref[...], w_ref[...], (((1,), (1,)), ((), ())),
                            preferred_element_type=jnp.float32)
        x = x + b_ref[...]
        mean = jnp.mean(x, axis=-1, keepdims=True)
        xc = x - mean
        var = jnp.mean(xc * xc, axis=-1, keepdims=True)
        xn = xc * lax.rsqrt(var + 1e-5)
        o_ref[...] = xn * g_ref[...] + be_ref[...]

    return pl.pallas_call(
        body,
        grid=(_N // nb,),
        in_specs=[
            pl.BlockSpec((nb, _D), lambda i: (i, 0)),
            pl.BlockSpec((_D, _D), lambda i: (0, 0)),
            pl.BlockSpec((1, _D), lambda i: (0, 0)),
            pl.BlockSpec((1, _D), lambda i: (0, 0)),
            pl.BlockSpec((1, _D), lambda i: (0, 0)),
        ],
        out_specs=pl.BlockSpec((nb, _D), lambda i: (i, 0)),
        out_shape=jax.ShapeDtypeStruct((_N, _D), jnp.float32),
    )(combined, w, b2, g2, be2)


def kernel(h3_5, h3_6, h3_7, h3_8, s2_11, s2_13, s2_14, s2_15,
           tab0, tab1, tab2, tab3, tab4, tab5, tab6, tab7,
           W, b, gamma, beta):
    idx_list = _tc_stage_idx((h3_5, h3_6, h3_7, h3_8, s2_11, s2_13, s2_14,
                              s2_15))
    combined = _sc_gather(idx_list, (tab0, tab1, tab2, tab3, tab4, tab5,
                                     tab6, tab7))
    out = _tc_fuse(combined, W, b.reshape(1, _D), gamma.reshape(1, _D),
                   beta.reshape(1, _D))
    return out.reshape(_B, _L, _D)


# speedup vs baseline: 27.7804x; 1.0474x over previous
import functools

import jax
import jax.numpy as jnp
from jax import lax
from jax.experimental import pallas as pl
from jax.experimental.pallas import tpu as pltpu
from jax.experimental.pallas import tpu_sc as plsc

_B, _L = 16384, 50
_N = _B * _L
_D = 128
_DL = 16
_NT = 8

_NC, _NS = 2, 16
_NW = _NC * _NS
_TPW = _N // _NW
_C = 128
_G = _TPW // _C
_GC = 20


def _tc_stage_idx(idx_arrs):
    nr = 800

    def body(*refs):
        for t in range(_NT):
            refs[_NT + t][...] = refs[t][...]

    return pl.pallas_call(
        body,
        grid=(_N // _C // nr,),
        in_specs=[pl.BlockSpec((nr, _C), lambda i: (i, 0))] * _NT,
        out_specs=[pl.BlockSpec((nr, _C), lambda i: (i, 0))] * _NT,
        out_shape=[jax.ShapeDtypeStruct((_N // _C, _C), jnp.int32)] * _NT,
    )(*[a.reshape(_N // _C, _C) for a in idx_arrs])


def _sc_gather(idx_list, tabs):
    mesh = plsc.VectorSubcoreMesh(core_axis_name="c", subcore_axis_name="s")

    @functools.partial(
        pl.kernel,
        out_type=jax.ShapeDtypeStruct((_N, _D), jnp.float32),
        mesh=mesh,
        scratch_types=[
            pltpu.VMEM((_NT, _GC, _C), jnp.int32),
            pltpu.VMEM((_NT, _C, _DL), jnp.float32),
            pltpu.VMEM((_NT, _C, _DL), jnp.float32),
            pltpu.SemaphoreType.DMA((_NT,)),
            pltpu.SemaphoreType.DMA((_NT,)),
            pltpu.SemaphoreType.DMA((_NT,)),
        ],
        compiler_params=pltpu.CompilerParams(use_tc_tiling_on_sc=False),
    )
    def k(i0, i1, i2, i3, i4, i5, i6, i7, t0, t1, t2, t3, t4, t5, t6, t7,
          out_hbm, idx_v, rows_a, rows_b, gsem, wsem_a, wsem_b):
        idx_refs = (i0, i1, i2, i3, i4, i5, i6, i7)
        tab_refs = (t0, t1, t2, t3, t4, t5, t6, t7)
        wid = lax.axis_index("s") * _NC + lax.axis_index("c")
        wrow = wid * _G
        wbase = wid * _TPW

        def drain_writes(rows_v, wsem):
            for t in range(_NT):
                pltpu.make_async_copy(
                    rows_v.at[t],
                    out_hbm.at[pl.ds(0, _C), pl.ds(t * _DL, _DL)],
                    wsem.at[t]).wait()

        def do_chunk(j, rows_v, wsem):
            g = j // _GC
            jc = j % _GC
            for t in range(_NT):
                pltpu.async_copy(tab_refs[t].at[idx_v.at[t, jc]],
                                 rows_v.at[t], gsem.at[t])
            for t in range(_NT):
                pltpu.make_async_copy(tab_refs[t].at[idx_v.at[t, jc]],
                                      rows_v.at[t], gsem.at[t]).wait()
            base = wbase + j * _C
            for t in range(_NT):
                pltpu.async_copy(
                    rows_v.at[t],
                    out_hbm.at[pl.ds(base, _C), pl.ds(t * _DL, _DL)],
                    wsem.at[t])

        def body(i, carry):
            j = 2 * i

            @pl.when(j % _GC == 0)
            def _idx():
                g = j // _GC
                for t in range(_NT):
                    pltpu.sync_copy(
                        idx_refs[t].at[pl.ds(wrow + g * _GC, _GC)],
                        idx_v.at[t])

            @pl.when(i > 0)
            def _da():
                drain_writes(rows_a, wsem_a)

            do_chunk(j, rows_a, wsem_a)

            @pl.when(i > 0)
            def _db():
                drain_writes(rows_b, wsem_b)

            do_chunk(j + 1, rows_b, wsem_b)
            return carry

        lax.fori_loop(0, _G // 2, body, 0)
        drain_writes(rows_a, wsem_a)
        drain_writes(rows_b, wsem_b)

    return k(*idx_list, *tabs)


def _tc_fuse(combined, w, b2, g2, be2):
    nb = 256

    def body(c_ref, w_ref, b_ref, g_ref, be_ref, o_ref):
        c = c_ref[...].reshape(nb * 8, _D)
        x = lax.dot_general(c, w_ref[...], (((1,), (1,)), ((), ())),
                            preferred_element_type=jnp.float32)
        x = x + b_ref[...]
        mean = jnp.mean(x, axis=-1, keepdims=True)
        xc = x - mean
        var = jnp.mean(xc * xc, axis=-1, keepdims=True)
        xn = xc * lax.rsqrt(var + 1e-5)
        o_ref[...] = xn * g_ref[...] + be_ref[...]

    return pl.pallas_call(
        body,
        grid=(_N // (nb * 8),),
        in_specs=[
            pl.BlockSpec((nb, 8, _D), lambda i: (i, 0, 0)),
            pl.BlockSpec((_D, _D), lambda i: (0, 0)),
            pl.BlockSpec((1, _D), lambda i: (0, 0)),
            pl.BlockSpec((1, _D), lambda i: (0, 0)),
            pl.BlockSpec((1, _D), lambda i: (0, 0)),
        ],
        out_specs=pl.BlockSpec((nb * 8, _D), lambda i: (i, 0)),
        out_shape=jax.ShapeDtypeStruct((_N, _D), jnp.float32),
    )(combined.reshape(_N // 8, 8, _D), w, b2, g2, be2)


def kernel(h3_5, h3_6, h3_7, h3_8, s2_11, s2_13, s2_14, s2_15,
           tab0, tab1, tab2, tab3, tab4, tab5, tab6, tab7,
           W, b, gamma, beta):
    idx_list = _tc_stage_idx((h3_5, h3_6, h3_7, h3_8, s2_11, s2_13, s2_14,
                              s2_15))
    combined = _sc_gather(idx_list, (tab0, tab1, tab2, tab3, tab4, tab5,
                                     tab6, tab7))
    out = _tc_fuse(combined, W, b.reshape(1, _D), gamma.reshape(1, _D),
                   beta.reshape(1, _D))
    return out.reshape(_B, _L, _D)
